# speedup vs baseline: 7.6674x; 7.6674x over previous
import functools

import jax
import jax.numpy as jnp
from jax import lax
from jax.experimental import pallas as pl
from jax.experimental.pallas import tpu as pltpu
from jax.experimental.pallas import tpu_sc as plsc

D_LFA = 128
D_FEAT = 128
K = 32

NC = 2
NS = 16
L = 16
NW = NC * NS
NPW = 320
G = 4
ROWS = G * K
GROUPS = NPW // G
N_PAD = NW * NPW


def _sc_gather_mean(features2d, idx_grouped):
    mesh = plsc.VectorSubcoreMesh(
        core_axis_name="c", subcore_axis_name="s", num_cores=NC, num_subcores=NS
    )

    @functools.partial(
        pl.kernel,
        out_type=jax.ShapeDtypeStruct((N_PAD, D_FEAT), jnp.float32),
        mesh=mesh,
        scratch_types=[
            pltpu.VMEM((GROUPS, ROWS), jnp.int32),
            pltpu.VMEM((ROWS, D_FEAT), jnp.float32),
            pltpu.VMEM((ROWS, D_FEAT), jnp.float32),
            pltpu.VMEM((G, D_FEAT), jnp.float32),
            pltpu.SemaphoreType.DMA,
            pltpu.SemaphoreType.DMA,
        ],
    )
    def gather_mean(feat_hbm, idx_hbm, out_hbm, idx_v, buf0, buf1, acc_v,
                    sem0, sem1):
        wid = lax.axis_index("s") * NC + lax.axis_index("c")
        pltpu.sync_copy(idx_hbm.at[pl.ds(wid * GROUPS, GROUPS)], idx_v)
        pltpu.async_copy(feat_hbm.at[idx_v.at[0]], buf0, sem0)

        nchunks = D_FEAT // L

        def process(g, buf):
            def node(i, carry):
                base = i * K
                accs = [jnp.zeros((L,), jnp.float32) for _ in range(nchunks)]
                for kk in range(K):
                    for c in range(nchunks):
                        accs[c] = accs[c] + buf[base + kk, pl.ds(c * L, L)]
                for c in range(nchunks):
                    acc_v[i, pl.ds(c * L, L)] = accs[c] * (1.0 / K)
                return carry
            lax.fori_loop(0, G, node, 0)
            pltpu.sync_copy(acc_v, out_hbm.at[pl.ds(wid * NPW + g * G, G)])

        def body(gg, carry):
            g0 = 2 * gg
            g1 = g0 + 1
            pltpu.async_copy(feat_hbm.at[idx_v.at[g1]], buf1, sem1)
            pltpu.make_async_copy(feat_hbm.at[idx_v.at[g0]], buf0, sem0).wait()
            process(g0, buf0)

            @pl.when(g1 + 1 < GROUPS)
            def _():
                pltpu.async_copy(feat_hbm.at[idx_v.at[g1 + 1]], buf0, sem0)

            pltpu.make_async_copy(feat_hbm.at[idx_v.at[g1]], buf1, sem1).wait()
            process(g1, buf1)
            return carry

        lax.fori_loop(0, GROUPS // 2, body, 0)

    return gather_mean(features2d, idx_grouped)


def _tc_geom_mlp(geom2, w_pad, b2):
    n = geom2.shape[0]
    nb = 1000
    grid = n // nb

    def body(g_ref, w_ref, b_ref, o_ref):
        g = g_ref[...]
        w = w_ref[...]
        bb = b_ref[...]
        acc = jnp.zeros((nb, D_LFA), jnp.float32)
        for k in range(K):
            t = (g[:, 4 * k:4 * k + 1] * w[0:1, :]
                 + g[:, 4 * k + 1:4 * k + 2] * w[1:2, :]
                 + g[:, 4 * k + 2:4 * k + 3] * w[2:3, :]
                 + g[:, 4 * k + 3:4 * k + 4] * w[3:4, :]
                 + bb)
            acc = acc + jnp.where(t >= 0, t, 0.1 * t)
        o_ref[...] = acc * (1.0 / K)

    return pl.pallas_call(
        body,
        grid=(grid,),
        in_specs=[
            pl.BlockSpec((nb, K * 4), lambda i: (i, 0)),
            pl.BlockSpec((8, D_LFA), lambda i: (0, 0)),
            pl.BlockSpec((1, D_LFA), lambda i: (0, 0)),
        ],
        out_specs=pl.BlockSpec((nb, D_LFA), lambda i: (i, 0)),
        out_shape=jax.ShapeDtypeStruct((n, D_LFA), jnp.float32),
    )(geom2, w_pad, b2)


def kernel(features, geom_features, neighbor_indices, W, b):
    bsz, n, k_ = neighbor_indices.shape
    f2 = features.reshape(n, D_FEAT)
    g2 = geom_features.reshape(n, k_ * 4)
    idx = neighbor_indices.reshape(n * k_).astype(jnp.int32)
    idx_p = jnp.zeros((N_PAD * k_,), jnp.int32).at[: n * k_].set(idx)
    idx_grouped = idx_p.reshape(N_PAD // G, ROWS)

    part_b = _sc_gather_mean(f2, idx_grouped)[:n]

    w_pad = jnp.zeros((8, D_LFA), jnp.float32).at[:4].set(W)
    part_a = _tc_geom_mlp(g2, w_pad, b.reshape(1, D_LFA))

    out = jnp.concatenate([part_a, part_b], axis=-1)
    return out.reshape(bsz, n, D_LFA + D_FEAT)


# speedup vs baseline: 13.6958x; 1.7862x over previous
import functools

import jax
import jax.numpy as jnp
from jax import lax
from jax.experimental import pallas as pl
from jax.experimental.pallas import tpu as pltpu
from jax.experimental.pallas import tpu_sc as plsc

D_LFA = 128
D_FEAT = 128
K = 32

NC = 2
NS = 16
L = 16
NW = NC * NS
NPW = 320
G = 4
ROWS = G * K
GROUPS = NPW // G
N_PAD = NW * NPW


def _sc_gather_mean(features2d, idx_grouped):
    mesh = plsc.VectorSubcoreMesh(
        core_axis_name="c", subcore_axis_name="s", num_cores=NC, num_subcores=NS
    )

    @functools.partial(
        pl.kernel,
        out_type=jax.ShapeDtypeStruct((N_PAD, D_FEAT), jnp.float32),
        mesh=mesh,
        scratch_types=[
            pltpu.VMEM((GROUPS, ROWS), jnp.int32),
            pltpu.VMEM((ROWS, D_FEAT), jnp.float32),
            pltpu.VMEM((ROWS, D_FEAT), jnp.float32),
            pltpu.VMEM((G, D_FEAT), jnp.float32),
            pltpu.VMEM_SHARED(features2d.shape, jnp.float32),
            pltpu.SemaphoreType.DMA,
            pltpu.SemaphoreType.DMA,
        ],
    )
    def gather_mean(feat_hbm, idx_hbm, out_hbm, idx_v, buf0, buf1, acc_v,
                    feat_sh, sem0, sem1):
        wid = lax.axis_index("s") * NC + lax.axis_index("c")
        sid = lax.axis_index("s")

        @pl.when(sid == 0)
        def _():
            pltpu.sync_copy(feat_hbm, feat_sh)

        pltpu.sync_copy(idx_hbm.at[pl.ds(wid * GROUPS, GROUPS)], idx_v)
        plsc.subcore_barrier()
        pltpu.async_copy(feat_sh.at[idx_v.at[0]], buf0, sem0)

        nchunks = D_FEAT // L

        def process(g, buf):
            def node(i, carry):
                base = i * K
                accs = [jnp.zeros((L,), jnp.float32) for _ in range(nchunks)]
                for kk in range(K):
                    for c in range(nchunks):
                        accs[c] = accs[c] + buf[base + kk, pl.ds(c * L, L)]
                for c in range(nchunks):
                    acc_v[i, pl.ds(c * L, L)] = accs[c] * (1.0 / K)
                return carry
            lax.fori_loop(0, G, node, 0)
            pltpu.sync_copy(acc_v, out_hbm.at[pl.ds(wid * NPW + g * G, G)])

        def body(gg, carry):
            g0 = 2 * gg
            g1 = g0 + 1
            pltpu.async_copy(feat_sh.at[idx_v.at[g1]], buf1, sem1)
            pltpu.make_async_copy(feat_sh.at[idx_v.at[g0]], buf0, sem0).wait()
            process(g0, buf0)

            @pl.when(g1 + 1 < GROUPS)
            def _():
                pltpu.async_copy(feat_sh.at[idx_v.at[g1 + 1]], buf0, sem0)

            pltpu.make_async_copy(feat_sh.at[idx_v.at[g1]], buf1, sem1).wait()
            process(g1, buf1)
            return carry

        lax.fori_loop(0, GROUPS // 2, body, 0)

    return gather_mean(features2d, idx_grouped)


def _tc_geom_mlp(geom2, w_pad, b2):
    n = geom2.shape[0]
    nb = 1000
    grid = n // nb

    def body(g_ref, w_ref, b_ref, o_ref):
        g = g_ref[...]
        w = w_ref[...]
        bb = b_ref[...]
        acc = jnp.zeros((nb, D_LFA), jnp.float32)
        for k in range(K):
            t = (g[:, 4 * k:4 * k + 1] * w[0:1, :]
                 + g[:, 4 * k + 1:4 * k + 2] * w[1:2, :]
                 + g[:, 4 * k + 2:4 * k + 3] * w[2:3, :]
                 + g[:, 4 * k + 3:4 * k + 4] * w[3:4, :]
                 + bb)
            acc = acc + jnp.where(t >= 0, t, 0.1 * t)
        o_ref[...] = acc * (1.0 / K)

    return pl.pallas_call(
        body,
        grid=(grid,),
        in_specs=[
            pl.BlockSpec((nb, K * 4), lambda i: (i, 0)),
            pl.BlockSpec((8, D_LFA), lambda i: (0, 0)),
            pl.BlockSpec((1, D_LFA), lambda i: (0, 0)),
        ],
        out_specs=pl.BlockSpec((nb, D_LFA), lambda i: (i, 0)),
        out_shape=jax.ShapeDtypeStruct((n, D_LFA), jnp.float32),
    )(geom2, w_pad, b2)


def kernel(features, geom_features, neighbor_indices, W, b):
    bsz, n, k_ = neighbor_indices.shape
    f2 = features.reshape(n, D_FEAT)
    g2 = geom_features.reshape(n, k_ * 4)
    idx = neighbor_indices.reshape(n * k_).astype(jnp.int32)
    idx_p = jnp.zeros((N_PAD * k_,), jnp.int32).at[: n * k_].set(idx)
    idx_grouped = idx_p.reshape(N_PAD // G, ROWS)

    part_b = _sc_gather_mean(f2, idx_grouped)[:n]

    w_pad = jnp.zeros((8, D_LFA), jnp.float32).at[:4].set(W)
    part_a = _tc_geom_mlp(g2, w_pad, b.reshape(1, D_LFA))

    out = jnp.concatenate([part_a, part_b], axis=-1)
    return out.reshape(bsz, n, D_LFA + D_FEAT)


# speedup vs baseline: 24.2196x; 1.7684x over previous
import functools

import jax
import jax.numpy as jnp
from jax import lax
from jax.experimental import pallas as pl
from jax.experimental.pallas import tpu as pltpu
from jax.experimental.pallas import tpu_sc as plsc

D_LFA = 128
D_FEAT = 128
K = 32

NC = 2
NS = 16
L = 16
NW = NC * NS
NPW = 320
G = 4
ROWS = G * K
GROUPS = NPW // G
N_PAD = NW * NPW


def _sc_gather_mean(features2d, idx_grouped):
    mesh = plsc.VectorSubcoreMesh(
        core_axis_name="c", subcore_axis_name="s", num_cores=NC, num_subcores=NS
    )

    @functools.partial(
        pl.kernel,
        out_type=jax.ShapeDtypeStruct((N_PAD, D_FEAT), jnp.float32),
        mesh=mesh,
        scratch_types=[
            pltpu.VMEM((GROUPS, ROWS), jnp.int32),
            pltpu.VMEM((ROWS, D_FEAT), jnp.float32),
            pltpu.VMEM((ROWS, D_FEAT), jnp.float32),
            pltpu.VMEM((G, D_FEAT), jnp.float32),
            pltpu.VMEM_SHARED(features2d.shape, jnp.float32),
            pltpu.SemaphoreType.DMA,
            pltpu.SemaphoreType.DMA,
        ],
    )
    def gather_mean(feat_hbm, idx_hbm, out_hbm, idx_v, buf0, buf1, acc_v,
                    feat_sh, sem0, sem1):
        wid = lax.axis_index("s") * NC + lax.axis_index("c")
        sid = lax.axis_index("s")

        @pl.when(sid == 0)
        def _():
            pltpu.sync_copy(feat_hbm, feat_sh)

        pltpu.sync_copy(idx_hbm.at[pl.ds(wid * GROUPS, GROUPS)], idx_v)
        plsc.subcore_barrier()
        pltpu.async_copy(feat_sh.at[idx_v.at[0]], buf0, sem0)

        nchunks = D_FEAT // L

        def process(g, buf):
            def node(i, carry):
                base = i * K
                accs = [jnp.zeros((L,), jnp.float32) for _ in range(nchunks)]
                for kk in range(K):
                    for c in range(nchunks):
                        accs[c] = accs[c] + buf[base + kk, pl.ds(c * L, L)]
                for c in range(nchunks):
                    acc_v[i, pl.ds(c * L, L)] = accs[c] * (1.0 / K)
                return carry
            lax.fori_loop(0, G, node, 0)
            pltpu.sync_copy(acc_v, out_hbm.at[pl.ds(wid * NPW + g * G, G)])

        def body(gg, carry):
            g0 = 2 * gg
            g1 = g0 + 1
            pltpu.async_copy(feat_sh.at[idx_v.at[g1]], buf1, sem1)
            pltpu.make_async_copy(feat_sh.at[idx_v.at[g0]], buf0, sem0).wait()
            process(g0, buf0)

            @pl.when(g1 + 1 < GROUPS)
            def _():
                pltpu.async_copy(feat_sh.at[idx_v.at[g1 + 1]], buf0, sem0)

            pltpu.make_async_copy(feat_sh.at[idx_v.at[g1]], buf1, sem1).wait()
            process(g1, buf1)
            return carry

        lax.fori_loop(0, GROUPS // 2, body, 0)

    return gather_mean(features2d, idx_grouped)


KC = 8


def _tc_geom_mlp(geom2, w_bd, b_tiled):
    n = geom2.shape[0]
    nb = 1000
    grid = (n // nb, K // KC)

    def body(g_ref, w_ref, b_ref, o_ref):
        c = pl.program_id(1)
        t = jnp.dot(g_ref[...], w_ref[...], preferred_element_type=jnp.float32)
        t = t + b_ref[...]
        t = jnp.where(t >= 0, t, 0.1 * t)
        s = t[:, 0:D_LFA]
        for j in range(1, KC):
            s = s + t[:, j * D_LFA:(j + 1) * D_LFA]
        s = s * (1.0 / K)

        @pl.when(c == 0)
        def _():
            o_ref[...] = s

        @pl.when(c > 0)
        def _():
            o_ref[...] = o_ref[...] + s

    return pl.pallas_call(
        body,
        grid=grid,
        in_specs=[
            pl.BlockSpec((nb, K * 4), lambda i, c: (i, 0)),
            pl.BlockSpec((K * 4, KC * D_LFA), lambda i, c: (0, c)),
            pl.BlockSpec((1, KC * D_LFA), lambda i, c: (0, c)),
        ],
        out_specs=pl.BlockSpec((nb, D_LFA), lambda i, c: (i, 0)),
        out_shape=jax.ShapeDtypeStruct((n, D_LFA), jnp.float32),
    )(geom2, w_bd, b_tiled)


def kernel(features, geom_features, neighbor_indices, W, b):
    bsz, n, k_ = neighbor_indices.shape
    f2 = features.reshape(n, D_FEAT)
    g2 = geom_features.reshape(n, k_ * 4)
    idx = neighbor_indices.reshape(n * k_).astype(jnp.int32)
    idx_p = jnp.zeros((N_PAD * k_,), jnp.int32).at[: n * k_].set(idx)
    idx_grouped = idx_p.reshape(N_PAD // G, ROWS)

    part_b = _sc_gather_mean(f2, idx_grouped)[:n]

    w_bd = jax.scipy.linalg.block_diag(*([W] * k_))
    b_tiled = jnp.tile(b, (k_,)).reshape(1, k_ * D_LFA)
    part_a = _tc_geom_mlp(g2, w_bd, b_tiled)

    out = jnp.concatenate([part_a, part_b], axis=-1)
    return out.reshape(bsz, n, D_LFA + D_FEAT)


# speedup vs baseline: 25.2281x; 1.0416x over previous
import functools

import jax
import jax.numpy as jnp
from jax import lax
from jax.experimental import pallas as pl
from jax.experimental.pallas import tpu as pltpu
from jax.experimental.pallas import tpu_sc as plsc

D_LFA = 128
D_FEAT = 128
K = 32

NC = 2
NS = 16
L = 16
NW = NC * NS
NPW = 320
G = 4
ROWS = G * K
GROUPS = NPW // G
N_PAD = NW * NPW


def _sc_gather_mean(features2d, idx_grouped):
    mesh = plsc.VectorSubcoreMesh(
        core_axis_name="c", subcore_axis_name="s", num_cores=NC, num_subcores=NS
    )

    @functools.partial(
        pl.kernel,
        out_type=jax.ShapeDtypeStruct((N_PAD, D_FEAT), jnp.float32),
        mesh=mesh,
        scratch_types=[
            pltpu.VMEM((GROUPS, ROWS), jnp.int32),
            pltpu.VMEM((ROWS, D_FEAT), jnp.float32),
            pltpu.VMEM((ROWS, D_FEAT), jnp.float32),
            pltpu.VMEM((G, D_FEAT), jnp.float32),
            pltpu.VMEM((G, D_FEAT), jnp.float32),
            pltpu.VMEM_SHARED(features2d.shape, jnp.float32),
            pltpu.SemaphoreType.DMA,
            pltpu.SemaphoreType.DMA,
            pltpu.SemaphoreType.DMA,
            pltpu.SemaphoreType.DMA,
        ],
    )
    def gather_mean(feat_hbm, idx_hbm, out_hbm, idx_v, buf0, buf1, acc0, acc1,
                    feat_sh, sem0, sem1, semo0, semo1):
        wid = lax.axis_index("s") * NC + lax.axis_index("c")
        sid = lax.axis_index("s")

        n_tab = features2d.shape[0]
        rows_per_tile = (n_tab // NS) // 8 * 8
        rem = n_tab - rows_per_tile * NS
        stage = pl.ds(sid * rows_per_tile, rows_per_tile)
        pltpu.sync_copy(feat_hbm.at[stage], feat_sh.at[stage])
        if rem:
            @pl.when(sid == 0)
            def _():
                tail = pl.ds(NS * rows_per_tile, rem)
                pltpu.sync_copy(feat_hbm.at[tail], feat_sh.at[tail])

        pltpu.sync_copy(idx_hbm.at[pl.ds(wid * GROUPS, GROUPS)], idx_v)
        plsc.subcore_barrier()
        pltpu.async_copy(feat_sh.at[idx_v.at[0]], buf0, sem0)

        nchunks = D_FEAT // L

        def process(g, buf, acc, semo):
            @pl.when(g >= 2)
            def _():
                pltpu.make_async_copy(
                    acc, out_hbm.at[pl.ds(wid * NPW + g * G, G)], semo).wait()

            def node(i, carry):
                base = i * K
                accs = [jnp.zeros((L,), jnp.float32) for _ in range(nchunks)]
                for kk in range(K):
                    for c in range(nchunks):
                        accs[c] = accs[c] + buf[base + kk, pl.ds(c * L, L)]
                for c in range(nchunks):
                    acc[i, pl.ds(c * L, L)] = accs[c] * (1.0 / K)
                return carry
            lax.fori_loop(0, G, node, 0)
            pltpu.async_copy(acc, out_hbm.at[pl.ds(wid * NPW + g * G, G)], semo)

        def body(gg, carry):
            g0 = 2 * gg
            g1 = g0 + 1
            pltpu.async_copy(feat_sh.at[idx_v.at[g1]], buf1, sem1)
            pltpu.make_async_copy(feat_sh.at[idx_v.at[g0]], buf0, sem0).wait()
            process(g0, buf0, acc0, semo0)

            @pl.when(g1 + 1 < GROUPS)
            def _():
                pltpu.async_copy(feat_sh.at[idx_v.at[g1 + 1]], buf0, sem0)

            pltpu.make_async_copy(feat_sh.at[idx_v.at[g1]], buf1, sem1).wait()
            process(g1, buf1, acc1, semo1)
            return carry

        lax.fori_loop(0, GROUPS // 2, body, 0)
        pltpu.make_async_copy(
            acc0, out_hbm.at[pl.ds(wid * NPW, G)], semo0).wait()
        pltpu.make_async_copy(
            acc1, out_hbm.at[pl.ds(wid * NPW, G)], semo1).wait()

    return gather_mean(features2d, idx_grouped)


KC = 8


def _tc_geom_mlp(geom2, w_bd, b_tiled):
    n = geom2.shape[0]
    nb = 1000
    grid = (n // nb, K // KC)

    def body(g_ref, w_ref, b_ref, o_ref):
        c = pl.program_id(1)
        t = jnp.dot(g_ref[...], w_ref[...], preferred_element_type=jnp.float32)
        t = t + b_ref[...]
        t = jnp.where(t >= 0, t, 0.1 * t)
        s = t[:, 0:D_LFA]
        for j in range(1, KC):
            s = s + t[:, j * D_LFA:(j + 1) * D_LFA]
        s = s * (1.0 / K)

        @pl.when(c == 0)
        def _():
            o_ref[...] = s

        @pl.when(c > 0)
        def _():
            o_ref[...] = o_ref[...] + s

    return pl.pallas_call(
        body,
        grid=grid,
        in_specs=[
            pl.BlockSpec((nb, K * 4), lambda i, c: (i, 0)),
            pl.BlockSpec((K * 4, KC * D_LFA), lambda i, c: (0, c)),
            pl.BlockSpec((1, KC * D_LFA), lambda i, c: (0, c)),
        ],
        out_specs=pl.BlockSpec((nb, D_LFA), lambda i, c: (i, 0)),
        out_shape=jax.ShapeDtypeStruct((n, D_LFA), jnp.float32),
    )(geom2, w_bd, b_tiled)


def kernel(features, geom_features, neighbor_indices, W, b):
    bsz, n, k_ = neighbor_indices.shape
    f2 = features.reshape(n, D_FEAT)
    g2 = geom_features.reshape(n, k_ * 4)
    idx = neighbor_indices.reshape(n * k_).astype(jnp.int32)
    idx_p = jnp.zeros((N_PAD * k_,), jnp.int32).at[: n * k_].set(idx)
    idx_grouped = idx_p.reshape(N_PAD // G, ROWS)

    part_b = _sc_gather_mean(f2, idx_grouped)[:n]

    w_bd = jax.scipy.linalg.block_diag(*([W] * k_))
    b_tiled = jnp.tile(b, (k_,)).reshape(1, k_ * D_LFA)
    part_a = _tc_geom_mlp(g2, w_bd, b_tiled)

    out = jnp.concatenate([part_a, part_b], axis=-1)
    return out.reshape(bsz, n, D_LFA + D_FEAT)
